# baseline (speedup 1.0000x reference)
import functools

import jax
import jax.numpy as jnp
from jax import lax
from jax.experimental import pallas as pl
from jax.experimental.pallas import tpu as pltpu
from jax.experimental.pallas import tpu_sc as plsc

DIM = 64
SEQ = 128
EPS = 1e-5
NC = 2
NS = 16
NW = NC * NS
CHUNK = 128
NBUF = 4
OBUF = 2
SPAD = SEQ + 1


def _rsqrt(x):
    i = plsc.bitcast(x, jnp.int32)
    i = jnp.int32(0x5F3759DF) - lax.shift_right_logical(i, 1)
    y = plsc.bitcast(i, jnp.float32)
    hx = 0.5 * x
    for _ in range(2):
        y = y * (1.5 - hx * y * y)
    return y


def _lane_sum(v):
    idx = lax.iota(jnp.int32, 16)
    dnums = lax.GatherDimensionNumbers(
        offset_dims=(), collapsed_slice_dims=(0,), start_index_map=(0,))
    for k in (8, 4, 2, 1):
        perm = lax.bitwise_xor(idx, jnp.int32(k))
        v = v + lax.gather(v, perm[:, None], dnums, slice_sizes=(1,),
                           mode=lax.GatherScatterMode.PROMISE_IN_BOUNDS)
    return v


DSLAB = 128
DPAD = DSLAB + 1
NFULL = 1000000 // DSLAB
NTAIL = 1000000 - NFULL * DSLAB


def _detile_body(tt_hbm, tail_hbm, out_hbm, vin, vout, vtail, in_sems,
                 out_sems):
    wid = lax.axis_index("s") * NC + lax.axis_index("c")

    half = lax.iota(jnp.int32, 16) // 2
    colp = (lax.iota(jnp.int32, 16) & 1) * DIM
    rows_tb = [tb * 8 + half for tb in range(DSLAB // 16)]

    niter = (NFULL + NW - 1) // NW

    def start_in(s, slot):
        pltpu.async_copy(tt_hbm.at[:, pl.ds(s * DSLAB, DSLAB)],
                         vin.at[slot], in_sems.at[slot])

    def wait_sem(dst, sem):
        pltpu.make_async_copy(out_hbm.at[pl.ds(0, dst.shape[0])],
                              dst.at[:, pl.ds(0, DSLAB)]
                              if dst.shape[1] == DPAD else dst,
                              sem).wait()

    def slab_id(i):
        return wid + NW * i

    @pl.when(slab_id(0) < NFULL)
    def _():
        start_in(slab_id(0), 0)

    @pl.when(slab_id(1) < NFULL)
    def _():
        start_in(slab_id(1), 1)

    def detile_group(g, _):
        for slot in range(2):
            i = g * 2 + slot
            s = slab_id(i)

            @pl.when(s < NFULL)
            def _():
                pltpu.make_async_copy(
                    out_hbm.at[pl.ds(0, DIM)], vin.at[slot],
                    in_sems.at[slot]).wait()

                @pl.when(i >= 2)
                def _():
                    wait_sem(vout.at[slot], out_sems.at[slot])

                def tb_body(tb, _):
                    rows = tb * 8 + half

                    @plsc.parallel_loop(0, DIM, unroll=8)
                    def d_body(d):
                        x = vin[slot, d, pl.ds(tb * 16, 16)]
                        plsc.store_scatter(vout.at[slot],
                                           [rows, colp + d], x)
                    return 0

                lax.fori_loop(0, DSLAB // 16, tb_body, 0)
                pltpu.async_copy(
                    vout.at[slot].at[:, pl.ds(0, DSLAB)],
                    out_hbm.at[pl.ds(s * (DSLAB // 2), DSLAB // 2)],
                    out_sems.at[slot])
                nxt = slab_id(i + 2)

                @pl.when(nxt < NFULL)
                def _():
                    start_in(nxt, slot)
        return 0

    lax.fori_loop(0, (niter + 1) // 2, detile_group, 0)

    for slot in range(2):
        @pl.when(slab_id(slot) < NFULL)
        def _():
            wait_sem(vout.at[slot], out_sems.at[slot])

    @pl.when(wid == 0)
    def _():
        pltpu.sync_copy(tail_hbm, vtail)
        pltpu.sync_copy(vtail,
                        out_hbm.at[pl.ds(NFULL * (DSLAB // 2),
                                         NTAIL // 2)])


def _sc_body(total_rows, ids_hbm, table_hbm, pos_hbm, gamma_hbm, beta_hbm,
             out_hbm, idx_v, rows_v, stage_v, pos_v, gamma_v, beta_v,
             in_sems, out_sems):
    wid = lax.axis_index("s") * NC + lax.axis_index("c")
    rows_per_w = total_rows // NW
    base = pl.multiple_of(wid * rows_per_w, CHUNK)
    seq_base = base // SEQ

    pltpu.sync_copy(ids_hbm.at[pl.ds(base, rows_per_w)], idx_v)
    pltpu.sync_copy(pos_hbm, pos_v)
    pltpu.sync_copy(gamma_hbm, gamma_v)
    pltpu.sync_copy(beta_hbm, beta_v)

    g = [gamma_v[pl.ds(16 * j, 16)] for j in range(4)]
    b = [beta_v[pl.ds(16 * j, 16)] for j in range(4)]
    row_ids = [lax.iota(jnp.int32, 16) + 16 * j for j in range(4)]
    inv_d = jnp.float32(1.0 / DIM)

    nchunks = rows_per_w // CHUNK

    def start_gather(c, buf):
        off = pl.multiple_of(c * CHUNK, CHUNK)
        pltpu.async_copy(table_hbm.at[idx_v.at[pl.ds(off, CHUNK)]],
                         rows_v.at[buf], in_sems.at[buf])

    def wait_dma(dst, sem):
        pltpu.make_async_copy(table_hbm.at[pl.ds(0, dst.shape[0])], dst,
                              sem).wait()

    def wait_out(oi):
        pltpu.make_async_copy(
            out_hbm.at[pl.ds(0, DIM)],
            stage_v.at[oi].at[:, pl.ds(0, SEQ)], out_sems.at[oi]).wait()

    start_gather(0, 0)
    start_gather(1, 1)

    def group_body(grp, _):
        for bi in range(NBUF):
            c = grp * NBUF + bi
            oi = bi % OBUF
            buf = rows_v.at[bi]
            stage = stage_v.at[oi]

            wait_dma(buf, in_sems.at[bi])

            @pl.when(c >= OBUF)
            def _():
                wait_out(oi)

            @plsc.parallel_loop(0, CHUNK, unroll=4)
            def row_body(r):
                x = [buf[r, pl.ds(16 * j, 16)] + pos_v[r, pl.ds(16 * j, 16)]
                     for j in range(4)]
                s = _lane_sum(x[0] + x[1] + x[2] + x[3])
                q = _lane_sum(x[0] * x[0] + x[1] * x[1]
                              + x[2] * x[2] + x[3] * x[3])
                mean = s * inv_d
                var = q * inv_d - mean * mean
                rstd = _rsqrt(var + EPS)
                col = jnp.full((16,), r, jnp.int32)
                for j in range(4):
                    y = (x[j] - mean) * rstd * g[j] + b[j]
                    plsc.store_scatter(stage, [row_ids[j], col], y)

            pltpu.async_copy(stage.at[:, pl.ds(0, SEQ)],
                             out_hbm.at[pl.ds((seq_base + c) * DIM, DIM)],
                             out_sems.at[oi])

            nxt = c + 2

            @pl.when(nxt < nchunks)
            def _():
                start_gather(nxt, (bi + 2) % NBUF)
        return 0

    lax.fori_loop(0, nchunks // NBUF, group_body, 0)

    for oi in range(OBUF):
        wait_out(oi)


def kernel(input_ids, token_table, pos_table, gamma, beta):
    batch, seq = input_ids.shape
    total_rows = batch * seq
    vocab = token_table.shape[0]
    ids_flat = input_ids.reshape(total_rows).astype(jnp.int32)
    rows_per_w = total_rows // NW

    mesh = plsc.VectorSubcoreMesh(core_axis_name="c", subcore_axis_name="s")

    tt = token_table.T
    tail_pairs = token_table[NFULL * DSLAB:].reshape(NTAIL // 2, 2 * DIM)
    pairs = pl.kernel(
        _detile_body,
        out_type=jax.ShapeDtypeStruct((vocab // 2, 2 * DIM), jnp.float32),
        mesh=mesh,
        compiler_params=pltpu.CompilerParams(
            needs_layout_passes=False, use_tc_tiling_on_sc=True),
        scratch_types=[
            pltpu.VMEM((2, DIM, DSLAB), jnp.float32),
            pltpu.VMEM((2, DSLAB // 2, DPAD), jnp.float32),
            pltpu.VMEM((NTAIL // 2, 2 * DIM), jnp.float32),
            pltpu.SemaphoreType.DMA((2,)),
            pltpu.SemaphoreType.DMA((2,)),
        ],
    )(tt, tail_pairs)
    token_lin = pairs.reshape(vocab, DIM)
    out_t = pl.kernel(
        functools.partial(_sc_body, total_rows),
        out_type=jax.ShapeDtypeStruct((batch * DIM, SEQ), jnp.float32),
        mesh=mesh,
        compiler_params=pltpu.CompilerParams(
            needs_layout_passes=False, use_tc_tiling_on_sc=False),
        scratch_types=[
            pltpu.VMEM((rows_per_w,), jnp.int32),
            pltpu.VMEM((NBUF, CHUNK, DIM), jnp.float32),
            pltpu.VMEM((OBUF, DIM, SPAD), jnp.float32),
            pltpu.VMEM((SEQ, DIM), jnp.float32),
            pltpu.VMEM((DIM,), jnp.float32),
            pltpu.VMEM((DIM,), jnp.float32),
            pltpu.SemaphoreType.DMA((NBUF,)),
            pltpu.SemaphoreType.DMA((OBUF,)),
        ],
    )(ids_flat, token_lin, pos_table, gamma, beta)
    return out_t.reshape(batch, DIM, SEQ).transpose(0, 2, 1)


# speedup vs baseline: 1.0021x; 1.0021x over previous
import functools

import jax
import jax.numpy as jnp
from jax import lax
from jax.experimental import pallas as pl
from jax.experimental.pallas import tpu as pltpu
from jax.experimental.pallas import tpu_sc as plsc

DIM = 64
SEQ = 128
EPS = 1e-5
NC = 2
NS = 16
NW = NC * NS
CHUNK = 128
NBUF = 4
OBUF = 2
SPAD = SEQ + 1


def _rsqrt(x):
    i = plsc.bitcast(x, jnp.int32)
    i = jnp.int32(0x5F3759DF) - lax.shift_right_logical(i, 1)
    y = plsc.bitcast(i, jnp.float32)
    hx = 0.5 * x
    for _ in range(2):
        y = y * (1.5 - hx * y * y)
    return y


def _lane_sum(v):
    idx = lax.iota(jnp.int32, 16)
    dnums = lax.GatherDimensionNumbers(
        offset_dims=(), collapsed_slice_dims=(0,), start_index_map=(0,))
    for k in (8, 4, 2, 1):
        perm = lax.bitwise_xor(idx, jnp.int32(k))
        v = v + lax.gather(v, perm[:, None], dnums, slice_sizes=(1,),
                           mode=lax.GatherScatterMode.PROMISE_IN_BOUNDS)
    return v


DSLAB = 256
DPAD = 2 * DIM + 1
NFULL = 1000000 // DSLAB
NTAIL = 1000000 - NFULL * DSLAB


def _detile_body(tt_hbm, tail_hbm, out_hbm, vin, vout, vtail, in_sems,
                 out_sems):
    wid = lax.axis_index("s") * NC + lax.axis_index("c")

    half = lax.iota(jnp.int32, 16) // 2
    colp = (lax.iota(jnp.int32, 16) & 1) * DIM
    rows_tb = [tb * 8 + half for tb in range(DSLAB // 16)]

    niter = (NFULL + NW - 1) // NW

    def start_in(s, slot):
        pltpu.async_copy(tt_hbm.at[:, pl.ds(s * DSLAB, DSLAB)],
                         vin.at[slot], in_sems.at[slot])

    def wait_in(slot):
        pltpu.make_async_copy(tt_hbm.at[:, pl.ds(0, DSLAB)],
                              vin.at[slot], in_sems.at[slot]).wait()

    def wait_out(slot):
        pltpu.make_async_copy(out_hbm.at[pl.ds(0, DSLAB // 2)],
                              vout.at[slot].at[:, pl.ds(0, 2 * DIM)],
                              out_sems.at[slot]).wait()

    def slab_id(i):
        return wid + NW * i

    @pl.when(slab_id(0) < NFULL)
    def _():
        start_in(slab_id(0), 0)

    @pl.when(slab_id(1) < NFULL)
    def _():
        start_in(slab_id(1), 1)

    def detile_group(g, _):
        for slot in range(2):
            i = g * 2 + slot
            s = slab_id(i)

            @pl.when(s < NFULL)
            def _():
                wait_in(slot)

                @pl.when(i >= 2)
                def _():
                    wait_out(slot)

                @plsc.parallel_loop(0, (DSLAB // 16) * DIM,
                                    unroll=16)
                def td_body(k):
                    tb = k >> 6
                    d = k & 63
                    x = vin[slot, d, pl.ds(tb * 16, 16)]
                    plsc.store_scatter(vout.at[slot],
                                       [tb * 8 + half, colp + d], x)
                pltpu.async_copy(
                    vout.at[slot].at[:, pl.ds(0, 2 * DIM)],
                    out_hbm.at[pl.ds(s * (DSLAB // 2), DSLAB // 2)],
                    out_sems.at[slot])
                nxt = slab_id(i + 2)

                @pl.when(nxt < NFULL)
                def _():
                    start_in(nxt, slot)
        return 0

    lax.fori_loop(0, (niter + 1) // 2, detile_group, 0)

    for slot in range(2):
        @pl.when(slab_id(slot) < NFULL)
        def _():
            wait_out(slot)

    @pl.when(wid == 0)
    def _():
        pltpu.sync_copy(tail_hbm, vtail)
        pltpu.sync_copy(vtail,
                        out_hbm.at[pl.ds(NFULL * (DSLAB // 2),
                                         NTAIL // 2)])


def _sc_body(total_rows, ids_hbm, table_hbm, pos_hbm, gamma_hbm, beta_hbm,
             out_hbm, idx_v, rows_v, stage_v, pos_v, gamma_v, beta_v,
             in_sems, out_sems):
    wid = lax.axis_index("s") * NC + lax.axis_index("c")
    rows_per_w = total_rows // NW
    base = pl.multiple_of(wid * rows_per_w, CHUNK)
    seq_base = base // SEQ

    pltpu.sync_copy(ids_hbm.at[pl.ds(base, rows_per_w)], idx_v)
    pltpu.sync_copy(pos_hbm, pos_v)
    pltpu.sync_copy(gamma_hbm, gamma_v)
    pltpu.sync_copy(beta_hbm, beta_v)

    g = [gamma_v[pl.ds(16 * j, 16)] for j in range(4)]
    b = [beta_v[pl.ds(16 * j, 16)] for j in range(4)]
    row_ids = [lax.iota(jnp.int32, 16) + 16 * j for j in range(4)]
    inv_d = jnp.float32(1.0 / DIM)

    nchunks = rows_per_w // CHUNK

    def start_gather(c, buf):
        off = pl.multiple_of(c * CHUNK, CHUNK)
        pltpu.async_copy(table_hbm.at[idx_v.at[pl.ds(off, CHUNK)]],
                         rows_v.at[buf], in_sems.at[buf])

    def wait_dma(dst, sem):
        pltpu.make_async_copy(table_hbm.at[pl.ds(0, dst.shape[0])], dst,
                              sem).wait()

    def wait_out(oi):
        pltpu.make_async_copy(
            out_hbm.at[pl.ds(0, DIM)],
            stage_v.at[oi].at[:, pl.ds(0, SEQ)], out_sems.at[oi]).wait()

    start_gather(0, 0)
    start_gather(1, 1)

    def group_body(grp, _):
        for bi in range(NBUF):
            c = grp * NBUF + bi
            oi = bi % OBUF
            buf = rows_v.at[bi]
            stage = stage_v.at[oi]

            wait_dma(buf, in_sems.at[bi])

            @pl.when(c >= OBUF)
            def _():
                wait_out(oi)

            @plsc.parallel_loop(0, CHUNK, unroll=4)
            def row_body(r):
                x = [buf[r, pl.ds(16 * j, 16)] + pos_v[r, pl.ds(16 * j, 16)]
                     for j in range(4)]
                s = _lane_sum(x[0] + x[1] + x[2] + x[3])
                q = _lane_sum(x[0] * x[0] + x[1] * x[1]
                              + x[2] * x[2] + x[3] * x[3])
                mean = s * inv_d
                var = q * inv_d - mean * mean
                rstd = _rsqrt(var + EPS)
                col = jnp.full((16,), r, jnp.int32)
                for j in range(4):
                    y = (x[j] - mean) * rstd * g[j] + b[j]
                    plsc.store_scatter(stage, [row_ids[j], col], y)

            pltpu.async_copy(stage.at[:, pl.ds(0, SEQ)],
                             out_hbm.at[pl.ds((seq_base + c) * DIM, DIM)],
                             out_sems.at[oi])

            nxt = c + 2

            @pl.when(nxt < nchunks)
            def _():
                start_gather(nxt, (bi + 2) % NBUF)
        return 0

    lax.fori_loop(0, nchunks // NBUF, group_body, 0)

    for oi in range(OBUF):
        wait_out(oi)


def kernel(input_ids, token_table, pos_table, gamma, beta):
    batch, seq = input_ids.shape
    total_rows = batch * seq
    vocab = token_table.shape[0]
    ids_flat = input_ids.reshape(total_rows).astype(jnp.int32)
    rows_per_w = total_rows // NW

    mesh = plsc.VectorSubcoreMesh(core_axis_name="c", subcore_axis_name="s")

    tt = token_table.T
    tail_pairs = token_table[NFULL * DSLAB:].reshape(NTAIL // 2, 2 * DIM)
    pairs = pl.kernel(
        _detile_body,
        out_type=jax.ShapeDtypeStruct((vocab // 2, 2 * DIM), jnp.float32),
        mesh=mesh,
        compiler_params=pltpu.CompilerParams(
            needs_layout_passes=False, use_tc_tiling_on_sc=True),
        scratch_types=[
            pltpu.VMEM((2, DIM, DSLAB), jnp.float32),
            pltpu.VMEM((2, DSLAB // 2, DPAD), jnp.float32),
            pltpu.VMEM((NTAIL // 2, 2 * DIM), jnp.float32),
            pltpu.SemaphoreType.DMA((2,)),
            pltpu.SemaphoreType.DMA((2,)),
        ],
    )(tt, tail_pairs)
    token_lin = pairs.reshape(vocab, DIM)
    out_t = pl.kernel(
        functools.partial(_sc_body, total_rows),
        out_type=jax.ShapeDtypeStruct((batch * DIM, SEQ), jnp.float32),
        mesh=mesh,
        compiler_params=pltpu.CompilerParams(
            needs_layout_passes=False, use_tc_tiling_on_sc=False),
        scratch_types=[
            pltpu.VMEM((rows_per_w,), jnp.int32),
            pltpu.VMEM((NBUF, CHUNK, DIM), jnp.float32),
            pltpu.VMEM((OBUF, DIM, SPAD), jnp.float32),
            pltpu.VMEM((SEQ, DIM), jnp.float32),
            pltpu.VMEM((DIM,), jnp.float32),
            pltpu.VMEM((DIM,), jnp.float32),
            pltpu.SemaphoreType.DMA((NBUF,)),
            pltpu.SemaphoreType.DMA((OBUF,)),
        ],
    )(ids_flat, token_lin, pos_table, gamma, beta)
    return out_t.reshape(batch, DIM, SEQ).transpose(0, 2, 1)
